# baseline (speedup 1.0000x reference)
import jax
import jax.numpy as jnp
from jax import lax
from jax.experimental import pallas as pl
from jax.experimental.pallas import tpu as pltpu
from jax.experimental.pallas import tpu_sc as plsc

_N = 10000
_E = 320000
_H = 128
_CH = 128
_NCH = 2560
_E_PAD = _NCH * _CH
_ACC_ROWS = 10240
_ZPT = _ACC_ROWS // 16
_OPT = 624
_BR = 1000


def _make_agg(F, feature_split, const_ones=False):
  mesh = plsc.VectorSubcoreMesh(core_axis_name="c", subcore_axis_name="s")
  cpt = (_NCH // 16) if feature_split else (_NCH // 32)
  n_groups = 4 if feature_split else 2

  def body(y_hbm, src_hbm, dst_hbm, out_hbm, src_v, dst_v, b0, b1, b2, b3,
           acc, g0, g1, g2, g3, s0, s1, s2, s3):
    bufs = (b0, b1, b2, b3)
    gsem = (g0, g1, g2, g3)
    ssem = (s0, s1, s2, s3)
    c = lax.axis_index("c")
    s = lax.axis_index("s")
    if feature_split:
      base = s * cpt
    else:
      base = c * (_NCH // 2) + s * cpt
    if not const_ones:
      pltpu.sync_copy(src_hbm.at[pl.ds(base, cpt)], src_v)
    pltpu.sync_copy(dst_hbm.at[pl.ds(base, cpt)], dst_v)

    def add_src_offset(off):
      offv = jnp.zeros((16,), jnp.int32) + off

      @pl.loop(0, cpt)
      def _addoff(i):
        for j in range(_CH // 16):
          src_v[i, pl.ds(j * 16, 16)] = src_v[i, pl.ds(j * 16, 16)] + offv

    def run_phase(out_group):
      @pl.loop(0, _CH)
      def _zero_fill(i):
        for j in range(F // 16):
          b0[i, pl.ds(j * 16, 16)] = jnp.zeros((16,), jnp.float32)

      z0 = s * _ZPT
      for r in range(_ZPT // _CH):
        pltpu.sync_copy(b0, acc.at[pl.ds(z0 + r * _CH, _CH)])
      plsc.subcore_barrier()

      if const_ones:
        @pl.loop(0, _CH)
        def _one_fill(i):
          for j in range(F // 16):
            b1[i, pl.ds(j * 16, 16)] = jnp.ones((16,), jnp.float32)

        for b in range(4):
          pltpu.async_copy(b1, acc.at[dst_v.at[b]], ssem[b], add=True)

        @pl.loop(0, (cpt - 4) // 4)
        def _cchunks(k):
          j0 = k * 4
          for b in range(4):
            pltpu.make_async_copy(b1, acc.at[dst_v.at[0]], ssem[b]).wait()
            pltpu.async_copy(b1, acc.at[dst_v.at[j0 + 4 + b]], ssem[b],
                             add=True)

        for b in range(4):
          pltpu.make_async_copy(b1, acc.at[dst_v.at[0]], ssem[b]).wait()
        for t in range(cpt % 4):
          pltpu.sync_copy(b1, acc.at[dst_v.at[cpt - cpt % 4 + t]], add=True)
        plsc.subcore_barrier()
      else:
        for b in range(4):
          pltpu.async_copy(y_hbm.at[src_v.at[b]], bufs[b], gsem[b])

        @pl.loop(0, cpt // 4)
        def _chunks(k):
          j0 = k * 4
          for b in range(4):
            pltpu.make_async_copy(y_hbm.at[src_v.at[0]], bufs[b],
                                  gsem[b]).wait()
            pltpu.async_copy(bufs[b], acc.at[dst_v.at[j0 + b]], ssem[b],
                             add=True)
          for b in range(4):

            @pl.when(j0 + b + 4 < cpt)
            def _():
              pltpu.make_async_copy(bufs[b], acc.at[dst_v.at[0]],
                                    ssem[b]).wait()
              pltpu.async_copy(y_hbm.at[src_v.at[j0 + b + 4]], bufs[b],
                               gsem[b])

        for b in range(4):
          pltpu.make_async_copy(bufs[b], acc.at[dst_v.at[0]],
                                ssem[b]).wait()
        for t in range(cpt % 4):
          pltpu.make_async_copy(y_hbm.at[src_v.at[0]], bufs[t],
                                gsem[t]).wait()
          pltpu.sync_copy(bufs[t], acc.at[dst_v.at[cpt - cpt % 4 + t]],
                          add=True)
        plsc.subcore_barrier()
      o0 = s * _OPT
      pltpu.sync_copy(acc.at[pl.ds(o0, _OPT)],
                      out_hbm.at[out_group, pl.ds(o0, _OPT)])

      @pl.when(s == 15)
      def _tail():
        t0 = 16 * _OPT
        pltpu.sync_copy(acc.at[pl.ds(t0, _N - 16 * _OPT)],
                        out_hbm.at[out_group, pl.ds(t0, _N - 16 * _OPT)])

    if feature_split:
      add_src_offset(c * _N)
      run_phase(c)
      plsc.subcore_barrier()
      add_src_offset(2 * _N)
      run_phase(2 + c)
    else:
      run_phase(c)

  return pl.kernel(
      body,
      out_type=jax.ShapeDtypeStruct((n_groups, _N, F), jnp.float32),
      mesh=mesh,
      compiler_params=pltpu.CompilerParams(use_tc_tiling_on_sc=False),
      scratch_types=[
          pltpu.VMEM((cpt, _CH), jnp.int32),
          pltpu.VMEM((cpt, _CH), jnp.int32),
          pltpu.VMEM((_CH, F), jnp.float32),
          pltpu.VMEM((_CH, F), jnp.float32),
          pltpu.VMEM((_CH, F), jnp.float32),
          pltpu.VMEM((_CH, F), jnp.float32),
          pltpu.VMEM_SHARED((_ACC_ROWS, F), jnp.float32),
      ] + [pltpu.SemaphoreType.DMA] * 8,
  )


def _t_matmul(x, w):
  def body(xr, wr, o_ref):
    o_ref[...] = jnp.dot(xr[...], wr[...], preferred_element_type=jnp.float32)

  return pl.pallas_call(
      body,
      grid=(_N // _BR,),
      in_specs=[
          pl.BlockSpec((_BR, _H), lambda i: (i, 0)),
          pl.BlockSpec((_H, _H), lambda i: (0, 0)),
      ],
      out_specs=pl.BlockSpec((_BR, _H), lambda i: (i, 0)),
      out_shape=jax.ShapeDtypeStruct((_N, _H), jnp.float32),
  )(x, w)


def _t_scale(degp, xw):
  def body(dp, xwr, y_ref, dis_ref):
    v = dp[...]
    deg = v[0, :, 0:1] + v[1, :, 0:1] + 1.0
    dis = lax.rsqrt(deg)
    y = xwr[...] * dis
    y_ref[...] = jnp.stack(
        [y[:, 32 * g:32 * g + 32] for g in range(4)], axis=0)
    dis_ref[...] = dis

  return pl.pallas_call(
      body,
      grid=(_N // _BR,),
      in_specs=[
          pl.BlockSpec((2, _BR, 16), lambda i: (0, i, 0)),
          pl.BlockSpec((_BR, _H), lambda i: (i, 0)),
      ],
      out_specs=[
          pl.BlockSpec((4, _BR, 32), lambda i: (0, i, 0)),
          pl.BlockSpec((_BR, 1), lambda i: (i, 0)),
      ],
      out_shape=[
          jax.ShapeDtypeStruct((4, _N, 32), jnp.float32),
          jax.ShapeDtypeStruct((_N, 1), jnp.float32),
      ],
  )(degp, xw)


def _t_mid(p, y, dis, b, w, grouped_out):
  def body(pr, yr, dr, br, wr, o_ref):
    vp, vy = pr[...], yr[...]
    agg = jnp.concatenate([vp[g] + vy[g] for g in range(4)], axis=1)
    dis_v = dr[...]
    h = jnp.maximum(agg * dis_v + br[...], 0.0)
    hw = jnp.dot(h, wr[...], preferred_element_type=jnp.float32) * dis_v
    if grouped_out:
      o_ref[...] = jnp.stack(
          [hw[:, 32 * g:32 * g + 32] for g in range(4)], axis=0)
    else:
      o_ref[...] = hw

  fo = w.shape[1]
  if grouped_out:
    out_spec = pl.BlockSpec((4, _BR, 32), lambda i: (0, i, 0))
    out_shape = jax.ShapeDtypeStruct((4, _N, 32), jnp.float32)
  else:
    out_spec = pl.BlockSpec((_BR, fo), lambda i: (i, 0))
    out_shape = jax.ShapeDtypeStruct((_N, fo), jnp.float32)
  return pl.pallas_call(
      body,
      grid=(_N // _BR,),
      in_specs=[
          pl.BlockSpec((4, _BR, 32), lambda i: (0, i, 0)),
          pl.BlockSpec((4, _BR, 32), lambda i: (0, i, 0)),
          pl.BlockSpec((_BR, 1), lambda i: (i, 0)),
          pl.BlockSpec((1, _H), lambda i: (0, 0)),
          pl.BlockSpec((_H, fo), lambda i: (0, 0)),
      ],
      out_specs=out_spec,
      out_shape=out_shape,
  )(p, y, dis, b, w)


def _t_last(p, y, dis, b3p, wcp, bcr):
  def body(pr, yr, dr, br, wr, bcref, o_ref):
    v = pr[...]
    h = jnp.maximum((v[0] + v[1] + yr[...]) * dr[...] + br[...], 0.0)
    o_ref[...] = jnp.dot(h, wr[...],
                         preferred_element_type=jnp.float32) + bcref[...]

  return pl.pallas_call(
      body,
      grid=(_N // _BR,),
      in_specs=[
          pl.BlockSpec((2, _BR, 16), lambda i: (0, i, 0)),
          pl.BlockSpec((_BR, 16), lambda i: (i, 0)),
          pl.BlockSpec((_BR, 1), lambda i: (i, 0)),
          pl.BlockSpec((1, 16), lambda i: (0, 0)),
          pl.BlockSpec((16, 16), lambda i: (0, 0)),
          pl.BlockSpec((1, 16), lambda i: (0, 0)),
      ],
      out_specs=pl.BlockSpec((_BR, 16), lambda i: (i, 0)),
      out_shape=jax.ShapeDtypeStruct((_N, 16), jnp.float32),
  )(p, y, dis, b3p, wcp, bcr)


def kernel(x, edge_index, W1, b1, W2, b2, W3, b3, Wc, bc):
  src = edge_index[0]
  dst = edge_index[1]
  pad = _E_PAD - _E
  src_p = jnp.concatenate(
      [src, jnp.zeros((pad,), jnp.int32)]).reshape(_NCH, _CH)
  dst_p = jnp.concatenate(
      [dst, jnp.full((pad,), _N, jnp.int32)]).reshape(_NCH, _CH)

  agg16 = _make_agg(16, feature_split=False)
  agg32 = _make_agg(32, feature_split=True)
  deg_agg = _make_agg(16, feature_split=False, const_ones=True)

  degp = deg_agg(jnp.zeros((16, 16), jnp.float32), src_p, dst_p)
  xw1 = _t_matmul(x, W1)
  y1, dis = _t_scale(degp, xw1)
  p1 = agg32(y1.reshape(4 * _N, 32), src_p, dst_p)
  y2 = _t_mid(p1, y1, dis, b1.reshape(1, _H), W2, grouped_out=True)
  p2 = agg32(y2.reshape(4 * _N, 32), src_p, dst_p)
  w3p = jnp.pad(W3, ((0, 0), (0, 8)))
  y3 = _t_mid(p2, y2, dis, b2.reshape(1, _H), w3p, grouped_out=False)
  p3 = agg16(y3, src_p, dst_p)
  out = _t_last(p3, y3, dis,
                jnp.pad(b3, (0, 8)).reshape(1, 16),
                jnp.pad(Wc, ((0, 8), (0, 0))),
                bc.reshape(1, 16))
  return out


# speedup vs baseline: 1.0104x; 1.0104x over previous
import jax
import jax.numpy as jnp
from jax import lax
from jax.experimental import pallas as pl
from jax.experimental.pallas import tpu as pltpu
from jax.experimental.pallas import tpu_sc as plsc

_N = 10000
_E = 320000
_H = 128
_CH = 128
_NCH = 2560
_E_PAD = _NCH * _CH
_ACC_ROWS = 10240
_ZPT = _ACC_ROWS // 16
_OPT = 624
_BR = 2000


def _make_agg(F, feature_split, const_ones=False):
  mesh = plsc.VectorSubcoreMesh(core_axis_name="c", subcore_axis_name="s")
  cpt = (_NCH // 16) if feature_split else (_NCH // 32)
  n_groups = 4 if feature_split else 2

  def body(y_hbm, src_hbm, dst_hbm, out_hbm, src_v, dst_v, b0, b1, b2, b3,
           acc, g0, g1, g2, g3, s0, s1, s2, s3):
    bufs = (b0, b1, b2, b3)
    gsem = (g0, g1, g2, g3)
    ssem = (s0, s1, s2, s3)
    c = lax.axis_index("c")
    s = lax.axis_index("s")
    if feature_split:
      base = s * cpt
    else:
      base = c * (_NCH // 2) + s * cpt
    if not const_ones:
      pltpu.sync_copy(src_hbm.at[pl.ds(base, cpt)], src_v)
    pltpu.sync_copy(dst_hbm.at[pl.ds(base, cpt)], dst_v)

    def add_src_offset(off):
      offv = jnp.zeros((16,), jnp.int32) + off

      @pl.loop(0, cpt)
      def _addoff(i):
        for j in range(_CH // 16):
          src_v[i, pl.ds(j * 16, 16)] = src_v[i, pl.ds(j * 16, 16)] + offv

    def run_phase(out_group):
      @pl.loop(0, _CH)
      def _zero_fill(i):
        for j in range(F // 16):
          b0[i, pl.ds(j * 16, 16)] = jnp.zeros((16,), jnp.float32)

      z0 = s * _ZPT
      for r in range(_ZPT // _CH):
        pltpu.sync_copy(b0, acc.at[pl.ds(z0 + r * _CH, _CH)])
      plsc.subcore_barrier()

      if const_ones:
        @pl.loop(0, _CH)
        def _one_fill(i):
          for j in range(F // 16):
            b1[i, pl.ds(j * 16, 16)] = jnp.ones((16,), jnp.float32)

        for b in range(4):
          pltpu.async_copy(b1, acc.at[dst_v.at[b]], ssem[b], add=True)

        @pl.loop(0, (cpt - 4) // 4)
        def _cchunks(k):
          j0 = k * 4
          for b in range(4):
            pltpu.make_async_copy(b1, acc.at[dst_v.at[0]], ssem[b]).wait()
            pltpu.async_copy(b1, acc.at[dst_v.at[j0 + 4 + b]], ssem[b],
                             add=True)

        for b in range(4):
          pltpu.make_async_copy(b1, acc.at[dst_v.at[0]], ssem[b]).wait()
        for t in range(cpt % 4):
          pltpu.sync_copy(b1, acc.at[dst_v.at[cpt - cpt % 4 + t]], add=True)
        plsc.subcore_barrier()
      else:
        for b in range(4):
          pltpu.async_copy(y_hbm.at[src_v.at[b]], bufs[b], gsem[b])

        @pl.loop(0, cpt // 4)
        def _chunks(k):
          j0 = k * 4
          for b in range(4):
            pltpu.make_async_copy(y_hbm.at[src_v.at[0]], bufs[b],
                                  gsem[b]).wait()
            pltpu.async_copy(bufs[b], acc.at[dst_v.at[j0 + b]], ssem[b],
                             add=True)
          for b in range(4):

            @pl.when(j0 + b + 4 < cpt)
            def _():
              pltpu.make_async_copy(bufs[b], acc.at[dst_v.at[0]],
                                    ssem[b]).wait()
              pltpu.async_copy(y_hbm.at[src_v.at[j0 + b + 4]], bufs[b],
                               gsem[b])

        for b in range(4):
          pltpu.make_async_copy(bufs[b], acc.at[dst_v.at[0]],
                                ssem[b]).wait()
        for t in range(cpt % 4):
          pltpu.make_async_copy(y_hbm.at[src_v.at[0]], bufs[t],
                                gsem[t]).wait()
          pltpu.sync_copy(bufs[t], acc.at[dst_v.at[cpt - cpt % 4 + t]],
                          add=True)
        plsc.subcore_barrier()
      o0 = s * _OPT
      pltpu.sync_copy(acc.at[pl.ds(o0, _OPT)],
                      out_hbm.at[out_group, pl.ds(o0, _OPT)])

      @pl.when(s == 15)
      def _tail():
        t0 = 16 * _OPT
        pltpu.sync_copy(acc.at[pl.ds(t0, _N - 16 * _OPT)],
                        out_hbm.at[out_group, pl.ds(t0, _N - 16 * _OPT)])

    if feature_split:
      add_src_offset(c * _N)
      run_phase(c)
      plsc.subcore_barrier()
      add_src_offset(2 * _N)
      run_phase(2 + c)
    else:
      run_phase(c)

  return pl.kernel(
      body,
      out_type=jax.ShapeDtypeStruct((n_groups, _N, F), jnp.float32),
      mesh=mesh,
      compiler_params=pltpu.CompilerParams(use_tc_tiling_on_sc=False),
      scratch_types=[
          pltpu.VMEM((cpt, _CH), jnp.int32),
          pltpu.VMEM((cpt, _CH), jnp.int32),
          pltpu.VMEM((_CH, F), jnp.float32),
          pltpu.VMEM((_CH, F), jnp.float32),
          pltpu.VMEM((_CH, F), jnp.float32),
          pltpu.VMEM((_CH, F), jnp.float32),
          pltpu.VMEM_SHARED((_ACC_ROWS, F), jnp.float32),
      ] + [pltpu.SemaphoreType.DMA] * 8,
  )


def _t_matmul(x, w):
  def body(xr, wr, o_ref):
    o_ref[...] = jnp.dot(xr[...], wr[...], preferred_element_type=jnp.float32)

  return pl.pallas_call(
      body,
      grid=(_N // _BR,),
      in_specs=[
          pl.BlockSpec((_BR, _H), lambda i: (i, 0)),
          pl.BlockSpec((_H, _H), lambda i: (0, 0)),
      ],
      out_specs=pl.BlockSpec((_BR, _H), lambda i: (i, 0)),
      out_shape=jax.ShapeDtypeStruct((_N, _H), jnp.float32),
  )(x, w)


def _t_scale(degp, xw):
  def body(dp, xwr, y_ref, dis_ref):
    v = dp[...]
    deg = v[0, :, 0:1] + v[1, :, 0:1] + 1.0
    dis = lax.rsqrt(deg)
    y = xwr[...] * dis
    y_ref[...] = jnp.stack(
        [y[:, 32 * g:32 * g + 32] for g in range(4)], axis=0)
    dis_ref[...] = dis

  return pl.pallas_call(
      body,
      grid=(_N // _BR,),
      in_specs=[
          pl.BlockSpec((2, _BR, 16), lambda i: (0, i, 0)),
          pl.BlockSpec((_BR, _H), lambda i: (i, 0)),
      ],
      out_specs=[
          pl.BlockSpec((4, _BR, 32), lambda i: (0, i, 0)),
          pl.BlockSpec((_BR, 1), lambda i: (i, 0)),
      ],
      out_shape=[
          jax.ShapeDtypeStruct((4, _N, 32), jnp.float32),
          jax.ShapeDtypeStruct((_N, 1), jnp.float32),
      ],
  )(degp, xw)


def _t_mid(p, y, dis, b, w, grouped_out):
  def body(pr, yr, dr, br, wr, o_ref):
    vp, vy = pr[...], yr[...]
    agg = jnp.concatenate([vp[g] + vy[g] for g in range(4)], axis=1)
    dis_v = dr[...]
    h = jnp.maximum(agg * dis_v + br[...], 0.0)
    hw = jnp.dot(h, wr[...], preferred_element_type=jnp.float32) * dis_v
    if grouped_out:
      o_ref[...] = jnp.stack(
          [hw[:, 32 * g:32 * g + 32] for g in range(4)], axis=0)
    else:
      o_ref[...] = hw

  fo = w.shape[1]
  if grouped_out:
    out_spec = pl.BlockSpec((4, _BR, 32), lambda i: (0, i, 0))
    out_shape = jax.ShapeDtypeStruct((4, _N, 32), jnp.float32)
  else:
    out_spec = pl.BlockSpec((_BR, fo), lambda i: (i, 0))
    out_shape = jax.ShapeDtypeStruct((_N, fo), jnp.float32)
  return pl.pallas_call(
      body,
      grid=(_N // _BR,),
      in_specs=[
          pl.BlockSpec((4, _BR, 32), lambda i: (0, i, 0)),
          pl.BlockSpec((4, _BR, 32), lambda i: (0, i, 0)),
          pl.BlockSpec((_BR, 1), lambda i: (i, 0)),
          pl.BlockSpec((1, _H), lambda i: (0, 0)),
          pl.BlockSpec((_H, fo), lambda i: (0, 0)),
      ],
      out_specs=out_spec,
      out_shape=out_shape,
  )(p, y, dis, b, w)


def _t_last(p, y, dis, b3p, wcp, bcr):
  def body(pr, yr, dr, br, wr, bcref, o_ref):
    v = pr[...]
    h = jnp.maximum((v[0] + v[1] + yr[...]) * dr[...] + br[...], 0.0)
    o_ref[...] = jnp.dot(h, wr[...],
                         preferred_element_type=jnp.float32) + bcref[...]

  return pl.pallas_call(
      body,
      grid=(_N // _BR,),
      in_specs=[
          pl.BlockSpec((2, _BR, 16), lambda i: (0, i, 0)),
          pl.BlockSpec((_BR, 16), lambda i: (i, 0)),
          pl.BlockSpec((_BR, 1), lambda i: (i, 0)),
          pl.BlockSpec((1, 16), lambda i: (0, 0)),
          pl.BlockSpec((16, 16), lambda i: (0, 0)),
          pl.BlockSpec((1, 16), lambda i: (0, 0)),
      ],
      out_specs=pl.BlockSpec((_BR, 16), lambda i: (i, 0)),
      out_shape=jax.ShapeDtypeStruct((_N, 16), jnp.float32),
  )(p, y, dis, b3p, wcp, bcr)


def kernel(x, edge_index, W1, b1, W2, b2, W3, b3, Wc, bc):
  src = edge_index[0]
  dst = edge_index[1]
  pad = _E_PAD - _E
  src_p = jnp.concatenate(
      [src, jnp.zeros((pad,), jnp.int32)]).reshape(_NCH, _CH)
  dst_p = jnp.concatenate(
      [dst, jnp.full((pad,), _N, jnp.int32)]).reshape(_NCH, _CH)

  agg16 = _make_agg(16, feature_split=False)
  agg32 = _make_agg(32, feature_split=True)
  deg_agg = _make_agg(16, feature_split=False, const_ones=True)

  degp = deg_agg(jnp.zeros((16, 16), jnp.float32), src_p, dst_p)
  xw1 = _t_matmul(x, W1)
  y1, dis = _t_scale(degp, xw1)
  p1 = agg32(y1.reshape(4 * _N, 32), src_p, dst_p)
  y2 = _t_mid(p1, y1, dis, b1.reshape(1, _H), W2, grouped_out=True)
  p2 = agg32(y2.reshape(4 * _N, 32), src_p, dst_p)
  w3p = jnp.pad(W3, ((0, 0), (0, 8)))
  y3 = _t_mid(p2, y2, dis, b2.reshape(1, _H), w3p, grouped_out=False)
  p3 = agg16(y3, src_p, dst_p)
  out = _t_last(p3, y3, dis,
                jnp.pad(b3, (0, 8)).reshape(1, 16),
                jnp.pad(Wc, ((0, 8), (0, 0))),
                bc.reshape(1, 16))
  return out
